# baseline (speedup 1.0000x reference)
import jax
import jax.numpy as jnp
from jax import lax
from jax.experimental import pallas as pl
from jax.experimental.pallas import tpu as pltpu
from jax.experimental.pallas import tpu_sc as plsc

_SIZE = 768
_ROWS = 16384
_SMALL = 193
_NC = 2
_NS = 16
_NW = _NC * _NS
_L = 16
_S = 3328
_NB1 = 4
_NB2 = 4
_C1 = 16
_C2 = 32
_D1 = 800
_D2 = 224


def _sc_body(idx1_hbm, idx2_hbm, t1, t2, out1_hbm, out2_hbm,
             idx_v, r1_v, r2_v, o_v, sem0, sem1, sem2, sem3):
    wid = lax.axis_index("s") * _NC + lax.axis_index("c")
    lane = lax.iota(jnp.int32, _L)
    last = lane == (_L - 1)
    odd_add = (lane & 1) * _ROWS
    sems = (sem0, sem1, sem2, sem3)

    def run_task(idx_hbm, tbl, out_hbm, r_v, C, NB):
        D = tbl.shape[1]
        pw = out_hbm.shape[0] // _NW
        base = wid * pw
        nsc = pw // _S
        nch = _S // C

        def issue(goff, buf):
            pltpu.async_copy(tbl.at[idx_v.at[pl.ds(2 * goff * C, C)]],
                             r_v.at[buf, pl.ds(0, C)], sems[buf])
            pltpu.async_copy(tbl.at[idx_v.at[pl.ds(2 * goff * C + C, C)]],
                             r_v.at[buf, pl.ds(C, C)], sems[buf])

        def wait(buf):
            pltpu.make_async_copy(tbl.at[idx_v.at[pl.ds(0, C)]],
                                  r_v.at[buf, pl.ds(0, C)],
                                  sems[buf]).wait()
            pltpu.make_async_copy(tbl.at[idx_v.at[pl.ds(0, C)]],
                                  r_v.at[buf, pl.ds(C, C)],
                                  sems[buf]).wait()

        def compute(buf, ooff):
            @plsc.parallel_loop(0, C, 1, unroll=2)
            def pair_body(p):
                acc_hi = jnp.zeros((_L,), jnp.float32)
                acc_lo = jnp.zeros((_L,), jnp.float32)
                for j in range(D // 32):
                    ai = plsc.bitcast(r_v[buf, 2 * p, pl.ds(j * 32, 32)],
                                      jnp.int32)
                    wi = plsc.bitcast(r_v[buf, 2 * p + 1, pl.ds(j * 32, 32)],
                                      jnp.int32)
                    acc_hi = acc_hi + (plsc.bitcast(ai, jnp.float32)
                                       * plsc.bitcast(wi, jnp.float32))
                    acc_lo = acc_lo + (
                        plsc.bitcast(lax.shift_left(ai, 16), jnp.float32)
                        * plsc.bitcast(lax.shift_left(wi, 16), jnp.float32))
                cs = plsc.cumsum(acc_hi + acc_lo)
                plsc.store_scatter(o_v,
                                   [jnp.full((_L,), ooff + p, jnp.int32)],
                                   cs, mask=last)

        def sc_loop(sc, carry):
            soff = base + sc * _S
            pltpu.sync_copy(idx_hbm.at[pl.ds(2 * soff, 2 * _S)], idx_v)

            @plsc.parallel_loop(0, 2 * _S // _L, 1, unroll=4)
            def remap_body(g):
                v = idx_v[pl.ds(g * _L, _L)]
                idx_v[pl.ds(g * _L, _L)] = v + odd_add

            for b in range(NB):
                issue(b, b)

            def pipe_body(g, c):
                g0 = NB * g
                for b in range(NB):
                    wait(b)
                    compute(b, (g0 + b) * C)

                    @pl.when(g0 + b + NB < nch)
                    def _():
                        issue(g0 + b + NB, b)

                return c

            lax.fori_loop(0, nch // NB, pipe_body, 0)
            pltpu.sync_copy(o_v, out_hbm.at[pl.ds(soff, _S)])
            return carry

        lax.fori_loop(0, nsc, sc_loop, 0)

    run_task(idx1_hbm, t1, out1_hbm, r1_v, _C1, _NB1)
    run_task(idx2_hbm, t2, out2_hbm, r2_v, _C2, _NB2)


def _sc_dots(idx1, idx2, t1, t2):
    k = idx1.shape[0] // 2
    mesh = plsc.VectorSubcoreMesh(core_axis_name="c", subcore_axis_name="s",
                                  num_cores=_NC, num_subcores=_NS)
    f = pl.kernel(
        _sc_body,
        out_type=[jax.ShapeDtypeStruct((k,), jnp.float32),
                  jax.ShapeDtypeStruct((k,), jnp.float32)],
        mesh=mesh,
        scratch_types=[
            pltpu.VMEM((2 * _S,), jnp.int32),
            pltpu.VMEM((_NB1, 2 * _C1, _D1), jnp.bfloat16),
            pltpu.VMEM((_NB2, 2 * _C2, _D2), jnp.bfloat16),
            pltpu.VMEM((_S,), jnp.float32),
        ] + [pltpu.SemaphoreType.DMA] * 4,
        compiler_params=pltpu.CompilerParams(needs_layout_passes=False,
                                             use_tc_tiling_on_sc=False),
    )
    return f(idx1, idx2, t1, t2)


def _loss_body(z_ref, t_ref, z1_ref, t1_ref, o_ref):
    def bce(z, t):
        return (jnp.maximum(z, 0.0) - z * t
                + jnp.log1p(jnp.exp(-jnp.abs(z))))

    o_ref[0, 0] = (jnp.sum(bce(z_ref[...], t_ref[...]))
                   + jnp.sum(bce(z1_ref[...], t1_ref[...])))


def _bce_loss(z, t, z1, t1):
    k = z.shape[0]
    rows = k // 128
    f = pl.pallas_call(
        _loss_body,
        out_shape=jax.ShapeDtypeStruct((1, 1), jnp.float32),
        out_specs=pl.BlockSpec(memory_space=pltpu.SMEM),
    )
    out = f(z.reshape(rows, 128), t.reshape(rows, 128),
            z1.reshape(rows, 128), t1.reshape(rows, 128))
    return out[0, 0]


def kernel(rnn_output, non_text_indices, non_text_expected_output, seen_before,
           non_text_indices1, non_text_expected_output1, seen_before1, W, W1):
    r = rnn_output.reshape(_ROWS, _SIZE)
    ones = jnp.ones((_ROWS, 1), jnp.float32)
    pad = jnp.zeros((_ROWS, 31), jnp.float32)
    bf = jnp.bfloat16
    a1 = jnp.concatenate([r, ones, pad], axis=1).astype(bf)
    w1 = jnp.concatenate([W[:_ROWS], pad], axis=1).astype(bf)
    a2 = jnp.concatenate([r[:, _SIZE - (_SMALL - 1):], ones, pad],
                         axis=1).astype(bf)
    w2 = jnp.concatenate([W1[:_ROWS], pad], axis=1).astype(bf)
    t1 = jnp.concatenate([a1, w1], axis=0)
    t2 = jnp.concatenate([a2, w2], axis=0)

    final, final1 = _sc_dots(non_text_indices.reshape(-1),
                             non_text_indices1.reshape(-1), t1, t2)
    loss = _bce_loss(final, non_text_expected_output,
                     final1, non_text_expected_output1)
    return final, loss


# speedup vs baseline: 1.5519x; 1.5519x over previous
import jax
import jax.numpy as jnp
from jax import lax
from jax.experimental import pallas as pl
from jax.experimental.pallas import tpu as pltpu
from jax.experimental.pallas import tpu_sc as plsc

_SIZE = 768
_ROWS = 16384
_SMALL = 193
_NC = 2
_NS = 16
_NW = _NC * _NS
_L = 16
_S = 3328
_NB1 = 4
_NB2 = 4
_C1 = 16
_C2 = 64
_D1 = 800
_D2 = 224


def _sc_body(ia1_hbm, ib1_hbm, ia2_hbm, ib2_hbm, ta1, tb1, ta2, tb2,
             out1_hbm, out2_hbm,
             ia_v, ib_v, ra1_v, rb1_v, ra2_v, rb2_v, o_v,
             sem0, sem1, sem2, sem3):
    wid = lax.axis_index("s") * _NC + lax.axis_index("c")
    lane = lax.iota(jnp.int32, _L)
    last = lane == (_L - 1)
    sems = (sem0, sem1, sem2, sem3)

    def run_task(ia_hbm, ib_hbm, tbl_a, tbl_b, out_hbm, ra_v, rb_v, C, NB):
        D = tbl_a.shape[1]
        pw = out_hbm.shape[0] // _NW
        base = wid * pw
        nsc = pw // _S
        nch = _S // C

        def issue(goff, buf):
            sem = sems[buf]
            pltpu.async_copy(tbl_a.at[ia_v.at[pl.ds(goff * C, C)]],
                             ra_v.at[buf], sem)
            pltpu.async_copy(tbl_b.at[ib_v.at[pl.ds(goff * C, C)]],
                             rb_v.at[buf], sem)

        def wait(buf):
            sem = sems[buf]
            pltpu.make_async_copy(tbl_a.at[ia_v.at[pl.ds(0, C)]],
                                  ra_v.at[buf], sem).wait()
            pltpu.make_async_copy(tbl_b.at[ib_v.at[pl.ds(0, C)]],
                                  rb_v.at[buf], sem).wait()

        def compute(buf, ooff):
            @plsc.parallel_loop(0, C, 1, unroll=2)
            def pair_body(p):
                acc_hi = jnp.zeros((_L,), jnp.float32)
                acc_lo = jnp.zeros((_L,), jnp.float32)
                for j in range(D // 32):
                    ai = plsc.bitcast(ra_v[buf, p, pl.ds(j * 32, 32)],
                                      jnp.int32)
                    wi = plsc.bitcast(rb_v[buf, p, pl.ds(j * 32, 32)],
                                      jnp.int32)
                    acc_hi = acc_hi + (plsc.bitcast(ai, jnp.float32)
                                       * plsc.bitcast(wi, jnp.float32))
                    acc_lo = acc_lo + (
                        plsc.bitcast(lax.shift_left(ai, 16), jnp.float32)
                        * plsc.bitcast(lax.shift_left(wi, 16), jnp.float32))
                cs = plsc.cumsum(acc_hi + acc_lo)
                plsc.store_scatter(o_v,
                                   [jnp.full((_L,), ooff + p, jnp.int32)],
                                   cs, mask=last)

        def sc_loop(sc, carry):
            soff = base + sc * _S
            pltpu.sync_copy(ia_hbm.at[pl.ds(soff, _S)], ia_v)
            pltpu.sync_copy(ib_hbm.at[pl.ds(soff, _S)], ib_v)
            for b in range(NB):
                issue(b, b)

            def pipe_body(g, c):
                g0 = NB * g
                for b in range(NB):
                    wait(b)
                    compute(b, (g0 + b) * C)

                    @pl.when(g0 + b + NB < nch)
                    def _():
                        issue(g0 + b + NB, b)

                return c

            lax.fori_loop(0, nch // NB, pipe_body, 0)
            pltpu.sync_copy(o_v, out_hbm.at[pl.ds(soff, _S)])
            return carry

        lax.fori_loop(0, nsc, sc_loop, 0)

    run_task(ia1_hbm, ib1_hbm, ta1, tb1, out1_hbm, ra1_v, rb1_v, _C1, _NB1)
    run_task(ia2_hbm, ib2_hbm, ta2, tb2, out2_hbm, ra2_v, rb2_v, _C2, _NB2)


def _sc_dots(ia1, ib1, ia2, ib2, ta1, tb1, ta2, tb2):
    k = ia1.shape[0]
    mesh = plsc.VectorSubcoreMesh(core_axis_name="c", subcore_axis_name="s",
                                  num_cores=_NC, num_subcores=_NS)
    f = pl.kernel(
        _sc_body,
        out_type=[jax.ShapeDtypeStruct((k,), jnp.float32),
                  jax.ShapeDtypeStruct((k,), jnp.float32)],
        mesh=mesh,
        scratch_types=[
            pltpu.VMEM((_S,), jnp.int32),
            pltpu.VMEM((_S,), jnp.int32),
            pltpu.VMEM((_NB1, _C1, _D1), jnp.bfloat16),
            pltpu.VMEM((_NB1, _C1, _D1), jnp.bfloat16),
            pltpu.VMEM((_NB2, _C2, _D2), jnp.bfloat16),
            pltpu.VMEM((_NB2, _C2, _D2), jnp.bfloat16),
            pltpu.VMEM((_S,), jnp.float32),
        ] + [pltpu.SemaphoreType.DMA] * 4,
        compiler_params=pltpu.CompilerParams(needs_layout_passes=False,
                                             use_tc_tiling_on_sc=False),
    )
    return f(ia1, ib1, ia2, ib2, ta1, tb1, ta2, tb2)


def _loss_body(z_ref, t_ref, z1_ref, t1_ref, o_ref):
    def bce(z, t):
        return (jnp.maximum(z, 0.0) - z * t
                + jnp.log1p(jnp.exp(-jnp.abs(z))))

    o_ref[0, 0] = (jnp.sum(bce(z_ref[...], t_ref[...]))
                   + jnp.sum(bce(z1_ref[...], t1_ref[...])))


def _bce_loss(z, t, z1, t1):
    k = z.shape[0]
    rows = k // 128
    f = pl.pallas_call(
        _loss_body,
        out_shape=jax.ShapeDtypeStruct((1, 1), jnp.float32),
        out_specs=pl.BlockSpec(memory_space=pltpu.SMEM),
    )
    out = f(z.reshape(rows, 128), t.reshape(rows, 128),
            z1.reshape(rows, 128), t1.reshape(rows, 128))
    return out[0, 0]


def kernel(rnn_output, non_text_indices, non_text_expected_output, seen_before,
           non_text_indices1, non_text_expected_output1, seen_before1, W, W1):
    r = rnn_output.reshape(_ROWS, _SIZE)
    ones = jnp.ones((_ROWS, 1), jnp.float32)
    pad = jnp.zeros((_ROWS, 31), jnp.float32)
    bf = jnp.bfloat16
    a1 = jnp.concatenate([r, ones, pad], axis=1).astype(bf)
    w1 = jnp.concatenate([W[:_ROWS], pad], axis=1).astype(bf)
    a2 = jnp.concatenate([r[:, _SIZE - (_SMALL - 1):], ones, pad],
                         axis=1).astype(bf)
    w2 = jnp.concatenate([W1[:_ROWS], pad], axis=1).astype(bf)

    i0 = non_text_indices[:, 0]
    i1 = non_text_indices[:, 1]
    j0 = non_text_indices1[:, 0]
    j1 = non_text_indices1[:, 1]

    final, final1 = _sc_dots(i0, i1, j0, j1, a1, w1, a2, w2)
    loss = _bce_loss(final, non_text_expected_output,
                     final1, non_text_expected_output1)
    return final, loss
